# baseline (speedup 1.0000x reference)
import jax
import jax.numpy as jnp
from jax import lax
from jax.experimental import pallas as pl
from jax.experimental.pallas import tpu as pltpu
from jax.experimental.pallas import tpu_sc as plsc

_ROWS = 8192
_COLS = 2048
_NC = 2
_NS = 16
_NW = _NC * _NS

_TEC_ROWS = 4608
_T_RPW = _TEC_ROWS // _NW
_T_CROWS = 16
_T_NB = 3
_T_NCH = _T_RPW // _T_CROWS

_SCS_BASE = _TEC_ROWS
_S_RPW = (_ROWS - _TEC_ROWS) // _NC
_S_CROWS = 56
_S_NB = 3
_S_NCH = _S_RPW // _S_CROWS


def _ring(src_hbm, dst_hbm, slot, sin, sout, base, crows, nch, nb):
    def in_copy(j):
        return pltpu.make_async_copy(
            src_hbm.at[pl.ds(base + j * crows, crows), :],
            slot(j % nb), sin[j % nb])

    def out_copy(j):
        return pltpu.make_async_copy(
            slot(j % nb),
            dst_hbm.at[pl.ds(base + j * crows, crows), :], sout[j % nb])

    for b in range(min(nb, nch)):
        in_copy(b).start()
    for j in range(nch):
        if j >= nb:
            out_copy(j - nb).wait()
            in_copy(j).start()
        in_copy(j).wait()
        out_copy(j).start()
    for j in range(max(0, nch - nb), nch):
        out_copy(j).wait()


def _tec_body(src_hbm, dst_hbm, tec_buf, scs_buf):
    del scs_buf
    wid = lax.axis_index("s") * _NC + lax.axis_index("c")
    sid = lax.axis_index("s")
    base = wid * _T_RPW

    def scoped(*sems):
        _ring(src_hbm, dst_hbm, lambda b: tec_buf.at[sid, b],
              sems[:_T_NB], sems[_T_NB:], base, _T_CROWS, _T_NCH, _T_NB)

    pl.run_scoped(scoped, *([pltpu.SemaphoreType.DMA] * (2 * _T_NB)))


def _scs_body(src_hbm, dst_hbm, tec_buf, scs_buf):
    del tec_buf
    cid = lax.axis_index("c")
    base = _SCS_BASE + cid * _S_RPW

    def scoped(*sems):
        _ring(src_hbm, dst_hbm, lambda b: scs_buf.at[b],
              sems[:_S_NB], sems[_S_NB:], base, _S_CROWS, _S_NCH, _S_NB)

    pl.run_scoped(scoped, *([pltpu.SemaphoreType.DMA] * (2 * _S_NB)))


def kernel(inputs, pos_table):
    del inputs
    k = pl.kernel(
        [_tec_body, _scs_body],
        out_type=jax.ShapeDtypeStruct((_ROWS, _COLS), jnp.float32),
        mesh=[
            plsc.VectorSubcoreMesh(core_axis_name="c", subcore_axis_name="s"),
            plsc.ScalarSubcoreMesh(axis_name="c"),
        ],
        scratch_types=[
            pltpu.VMEM_SHARED((_NS, _T_NB, _T_CROWS, _COLS), jnp.float32),
            pltpu.VMEM_SHARED((_S_NB, _S_CROWS, _COLS), jnp.float32),
        ],
    )
    return k(pos_table)


# speedup vs baseline: 1.2085x; 1.2085x over previous
import jax
import jax.numpy as jnp
from jax import lax
from jax.experimental import pallas as pl
from jax.experimental.pallas import tpu as pltpu
from jax.experimental.pallas import tpu_sc as plsc

_ROWS = 8192
_COLS = 2048
_NC = 2
_NS = 16
_NW = _NC * _NS
_RPW = _ROWS // _NW
_CROWS = 16
_NB = 3
_NCH = _RPW // _CROWS


def _tec_body(src_hbm, dst_hbm, buf, *sems):
    sin = sems[:_NB]
    sout = sems[_NB:]
    wid = lax.axis_index("s") * _NC + lax.axis_index("c")
    sid = lax.axis_index("s")
    base = wid * _RPW

    def in_copy(j):
        return pltpu.make_async_copy(
            src_hbm.at[pl.ds(base + j * _CROWS, _CROWS), :],
            buf.at[sid, j % _NB], sin[j % _NB])

    def out_copy(j):
        return pltpu.make_async_copy(
            buf.at[sid, j % _NB],
            dst_hbm.at[pl.ds(base + j * _CROWS, _CROWS), :], sout[j % _NB])

    for b in range(_NB):
        in_copy(b).start()
    for j in range(_NCH):
        if j >= _NB:
            out_copy(j - _NB).wait()
            in_copy(j).start()
        in_copy(j).wait()
        out_copy(j).start()
    for j in range(_NCH - _NB, _NCH):
        out_copy(j).wait()


def kernel(inputs, pos_table):
    del inputs
    k = pl.kernel(
        _tec_body,
        out_type=jax.ShapeDtypeStruct((_ROWS, _COLS), jnp.float32),
        mesh=plsc.VectorSubcoreMesh(core_axis_name="c", subcore_axis_name="s"),
        scratch_types=(
            [pltpu.VMEM_SHARED((_NS, _NB, _CROWS, _COLS), jnp.float32)]
            + [pltpu.SemaphoreType.DMA] * (2 * _NB)
        ),
    )
    return k(pos_table)
